# Initial kernel scaffold; baseline (speedup 1.0000x reference)
import jax
import jax.numpy as jnp
from jax.experimental import pallas as pl


def kernel(h0, x, edges, edge_attr, node_mask, edge_mask, n_nodes, params):
    raise NotImplementedError("write your pallas kernel here")


# speedup vs baseline: 11.8368x; 11.8368x over previous
import functools

import jax
import jax.numpy as jnp
from jax.experimental import pallas as pl
from jax.experimental.pallas import tpu as pltpu

N = 10000
HID = 128
D_EDGE = 16
D_INNER = 256
D_STATE = 64
DT_RANK = 8
D_CONV = 4

SCAN_T = 400


def _silu(v):
    return v * jax.nn.sigmoid(v)


def _scan_body(xc_ref, dt_ref, bc_ref, z_ref, at_ref, d_ref, ow_ref, ob_ref,
               out_ref, carry_ref, ys_ref):
    i = pl.program_id(0)

    @pl.when(i == 0)
    def _():
        carry_ref[...] = jnp.zeros_like(carry_ref)

    at = at_ref[...]
    rows = jax.lax.broadcasted_iota(jnp.int32, (2 * D_STATE, 2 * D_STATE), 0)
    cols = jax.lax.broadcasted_iota(jnp.int32, (2 * D_STATE, 2 * D_STATE), 1)
    eye = (rows == cols).astype(jnp.float32)

    def step(t, carry):
        bc_row = bc_ref[pl.ds(t, 1), :]
        bc_b = jnp.broadcast_to(bc_row, (2 * D_STATE, 2 * D_STATE))
        col = jnp.sum(bc_b * eye, axis=1, keepdims=True)
        b_col = col[0:D_STATE, :]
        c_col = col[D_STATE:2 * D_STATE, :]
        dt_row = dt_ref[pl.ds(t, 1), :]
        x_row = xc_ref[pl.ds(t, 1), :]
        da = jnp.exp(at * dt_row)
        u = dt_row * x_row
        carry = carry * da + b_col * u
        y = jnp.sum(carry * c_col, axis=0, keepdims=True)
        ys_ref[pl.ds(t, 1), :] = y
        return carry

    carry = jax.lax.fori_loop(0, SCAN_T, step, carry_ref[...])
    carry_ref[...] = carry

    ys = ys_ref[...]
    xc = xc_ref[...]
    z = z_ref[...]
    y = (ys + xc * d_ref[...]) * _silu(z)
    out_ref[...] = jnp.dot(y, ow_ref[...],
                           preferred_element_type=jnp.float32) + ob_ref[...]


def _mamba_scan(xc, dt, b, c, z, a, d_vec, out_w, out_b):
    bc = jnp.concatenate([b, c], axis=1)
    at = a.T
    d_row = d_vec.reshape(1, D_INNER)
    ob_row = out_b.reshape(1, HID)
    grid = (N // SCAN_T,)
    return pl.pallas_call(
        _scan_body,
        grid=grid,
        in_specs=[
            pl.BlockSpec((SCAN_T, D_INNER), lambda i: (i, 0)),
            pl.BlockSpec((SCAN_T, D_INNER), lambda i: (i, 0)),
            pl.BlockSpec((SCAN_T, 2 * D_STATE), lambda i: (i, 0)),
            pl.BlockSpec((SCAN_T, D_INNER), lambda i: (i, 0)),
            pl.BlockSpec((D_STATE, D_INNER), lambda i: (0, 0)),
            pl.BlockSpec((1, D_INNER), lambda i: (0, 0)),
            pl.BlockSpec((D_INNER, HID), lambda i: (0, 0)),
            pl.BlockSpec((1, HID), lambda i: (0, 0)),
        ],
        out_specs=pl.BlockSpec((SCAN_T, HID), lambda i: (i, 0)),
        out_shape=jax.ShapeDtypeStruct((N, HID), jnp.float32),
        scratch_shapes=[
            pltpu.VMEM((D_STATE, D_INNER), jnp.float32),
            pltpu.VMEM((SCAN_T, D_INNER), jnp.float32),
        ],
    )(xc, dt, bc, z, at, d_row, out_w, ob_row)


def _mamba(lp, h):
    xz = h @ lp['in_w'] + lp['in_b']
    xm, z = jnp.split(xz, 2, axis=-1)
    n = xm.shape[0]
    xpad = jnp.pad(xm, ((D_CONV - 1, 0), (0, 0)))
    conv = jnp.zeros_like(xm)
    for k in range(D_CONV):
        conv = conv + xpad[k:k + n] * lp['conv_w'][:, k][None, :]
    xc = _silu(conv + lp['conv_b'][None, :])
    xdbl = xc @ lp['xproj_w']
    dt = jax.nn.softplus(xdbl[:, :DT_RANK] @ lp['dt_w'] + lp['dt_b'])
    b = xdbl[:, DT_RANK:DT_RANK + D_STATE]
    c = xdbl[:, DT_RANK + D_STATE:]
    a = -jnp.exp(lp['A_log'])
    return _mamba_scan(xc, dt, b, c, z, a, lp['D'], lp['out_w'], lp['out_b'])


def _layer(lp, h, h0, coord, src, dst, edge_attr, edge_mask):
    coord_diff = coord[src] - coord[dst]
    radial = jnp.sum(coord_diff ** 2, axis=1, keepdims=True)
    e_in = jnp.concatenate([h[src], h[dst], radial, edge_attr], axis=1)
    ef = _silu(e_in @ lp['edge_w1'] + lp['edge_b1'])
    ef = _silu(ef @ lp['edge_w2'] + lp['edge_b2'])
    mi = jax.ops.segment_sum(ef, src, num_segments=h.shape[0])
    hc = jnp.concatenate([mi, h, h0], axis=1)
    hn = _silu(hc @ lp['node_w1'] + lp['node_b1'])
    hn = hn @ lp['node_w2'] + lp['node_b2']
    mu = hn.mean(axis=-1, keepdims=True)
    var = ((hn - mu) ** 2).mean(axis=-1, keepdims=True)
    hn = (hn - mu) / jnp.sqrt(var + 1e-05) * lp['ln_g'] + lp['ln_b']
    hn = jnp.clip(hn, -10.0, 10.0)
    return _mamba(lp, hn)


def kernel(h0, x, edges, edge_attr, node_mask, edge_mask, n_nodes, params):
    src = edges[0]
    dst = edges[1]
    h = h0 @ params['emb_w'] + params['emb_b']
    for lp in params['layers']:
        h = _layer(lp, h, h0, x, src, dst, edge_attr, edge_mask)
    h = _silu(h @ params['dec_w1'] + params['dec_b1'])
    h = h @ params['dec_w2'] + params['dec_b2']
    h = h * node_mask
    h = h.reshape(-1, N, HID).sum(axis=1)
    g = _silu(h @ params['g_w1'] + params['g_b1'])
    pred = g @ params['g_w2'] + params['g_b2']
    return pred[:, 0]


# speedup vs baseline: 12.8769x; 1.0879x over previous
import functools

import jax
from jax import lax
import jax.numpy as jnp
from jax.experimental import pallas as pl
from jax.experimental.pallas import tpu as pltpu
from jax.experimental.pallas import tpu_sc as plsc

N = 10000
E = 160000
HID = 128
D_EDGE = 16
D_INNER = 256
D_STATE = 64
DT_RANK = 8
D_CONV = 4

SCAN_T = 400

TBL = 144
NC, NS = 2, 16
NW = NC * NS
GCHUNK = 128
G_PER_W = (2 * E + NW * GCHUNK - 1) // (NW * GCHUNK) * GCHUNK
G_PAD = NW * G_PER_W
S_PER_W = (E + NW * GCHUNK - 1) // (NW * GCHUNK) * GCHUNK
S_PAD = NW * S_PER_W
N_PER_T = N // NS


def _sc_gather_body(table, idx_hbm, out_hbm, idx_v, rows_v, sem):
    wid = lax.axis_index("c") * NS + lax.axis_index("s")
    base = wid * G_PER_W

    def chunk(j, _):
        off = base + j * GCHUNK
        pltpu.sync_copy(idx_hbm.at[pl.ds(off, GCHUNK)], idx_v)
        pltpu.async_copy(table.at[idx_v], rows_v, sem).wait()
        pltpu.sync_copy(rows_v, out_hbm.at[pl.ds(off, GCHUNK)])
        return _

    lax.fori_loop(0, G_PER_W // GCHUNK, chunk, 0, unroll=False)


def _sc_gather(table, idx_all):
    mesh = plsc.VectorSubcoreMesh(core_axis_name="c", subcore_axis_name="s")
    f = pl.kernel(
        _sc_gather_body,
        mesh=mesh,
        compiler_params=pltpu.CompilerParams(use_tc_tiling_on_sc=False),
        out_type=jax.ShapeDtypeStruct((G_PAD, TBL), jnp.float32),
        scratch_types=[
            pltpu.VMEM((GCHUNK,), jnp.int32),
            pltpu.VMEM((GCHUNK, TBL), jnp.float32),
            pltpu.SemaphoreType.DMA,
        ],
    )
    return f(table, idx_all)


def _sc_scatter_body(ef_hbm, idx_hbm, zeros_hbm, out_hbm, idx_v, ef_v, acc):
    cid = lax.axis_index("c")
    sid = lax.axis_index("s")
    wid = cid * NS + sid
    base = wid * S_PER_W

    @pl.when(sid == 0)
    def _():
        pltpu.sync_copy(zeros_hbm, acc)

    plsc.subcore_barrier()

    def chunk(j, _):
        off = base + j * GCHUNK
        pltpu.sync_copy(idx_hbm.at[pl.ds(off, GCHUNK)], idx_v)
        pltpu.sync_copy(ef_hbm.at[pl.ds(off, GCHUNK)], ef_v)
        pltpu.sync_copy(ef_v, acc.at[idx_v], add=True)
        return _

    lax.fori_loop(0, S_PER_W // GCHUNK, chunk, 0, unroll=False)
    plsc.subcore_barrier()
    pltpu.sync_copy(acc.at[pl.ds(sid * N_PER_T, N_PER_T)],
                    out_hbm.at[cid].at[pl.ds(sid * N_PER_T, N_PER_T)])


def _sc_segment_sum(ef_pad, src_pad, zeros):
    mesh = plsc.VectorSubcoreMesh(core_axis_name="c", subcore_axis_name="s")
    f = pl.kernel(
        _sc_scatter_body,
        mesh=mesh,
        compiler_params=pltpu.CompilerParams(use_tc_tiling_on_sc=False),
        out_type=jax.ShapeDtypeStruct((NC, N, HID), jnp.float32),
        scratch_types=[
            pltpu.VMEM((GCHUNK,), jnp.int32),
            pltpu.VMEM((GCHUNK, HID), jnp.float32),
            pltpu.VMEM_SHARED((N, HID), jnp.float32),
        ],
    )
    return f(ef_pad, src_pad, zeros)


def _silu(v):
    return v * jax.nn.sigmoid(v)


def _scan_body(xc_ref, dt_ref, bc_ref, z_ref, at_ref, d_ref, ow_ref, ob_ref,
               out_ref, carry_ref, ys_ref):
    i = pl.program_id(0)

    @pl.when(i == 0)
    def _():
        carry_ref[...] = jnp.zeros_like(carry_ref)

    at = at_ref[...]
    rows = jax.lax.broadcasted_iota(jnp.int32, (2 * D_STATE, 2 * D_STATE), 0)
    cols = jax.lax.broadcasted_iota(jnp.int32, (2 * D_STATE, 2 * D_STATE), 1)
    eye = (rows == cols).astype(jnp.float32)

    def step(t, carry):
        bc_row = bc_ref[pl.ds(t, 1), :]
        bc_b = jnp.broadcast_to(bc_row, (2 * D_STATE, 2 * D_STATE))
        col = jnp.sum(bc_b * eye, axis=1, keepdims=True)
        b_col = col[0:D_STATE, :]
        c_col = col[D_STATE:2 * D_STATE, :]
        dt_row = dt_ref[pl.ds(t, 1), :]
        x_row = xc_ref[pl.ds(t, 1), :]
        da = jnp.exp(at * dt_row)
        u = dt_row * x_row
        carry = carry * da + b_col * u
        y = jnp.sum(carry * c_col, axis=0, keepdims=True)
        ys_ref[pl.ds(t, 1), :] = y
        return carry

    carry = jax.lax.fori_loop(0, SCAN_T, step, carry_ref[...])
    carry_ref[...] = carry

    ys = ys_ref[...]
    xc = xc_ref[...]
    z = z_ref[...]
    y = (ys + xc * d_ref[...]) * _silu(z)
    out_ref[...] = jnp.dot(y, ow_ref[...],
                           preferred_element_type=jnp.float32) + ob_ref[...]


def _mamba_scan(xc, dt, b, c, z, a, d_vec, out_w, out_b):
    bc = jnp.concatenate([b, c], axis=1)
    at = a.T
    d_row = d_vec.reshape(1, D_INNER)
    ob_row = out_b.reshape(1, HID)
    grid = (N // SCAN_T,)
    return pl.pallas_call(
        _scan_body,
        grid=grid,
        in_specs=[
            pl.BlockSpec((SCAN_T, D_INNER), lambda i: (i, 0)),
            pl.BlockSpec((SCAN_T, D_INNER), lambda i: (i, 0)),
            pl.BlockSpec((SCAN_T, 2 * D_STATE), lambda i: (i, 0)),
            pl.BlockSpec((SCAN_T, D_INNER), lambda i: (i, 0)),
            pl.BlockSpec((D_STATE, D_INNER), lambda i: (0, 0)),
            pl.BlockSpec((1, D_INNER), lambda i: (0, 0)),
            pl.BlockSpec((D_INNER, HID), lambda i: (0, 0)),
            pl.BlockSpec((1, HID), lambda i: (0, 0)),
        ],
        out_specs=pl.BlockSpec((SCAN_T, HID), lambda i: (i, 0)),
        out_shape=jax.ShapeDtypeStruct((N, HID), jnp.float32),
        scratch_shapes=[
            pltpu.VMEM((D_STATE, D_INNER), jnp.float32),
            pltpu.VMEM((SCAN_T, D_INNER), jnp.float32),
        ],
    )(xc, dt, bc, z, at, d_row, out_w, ob_row)


def _mamba(lp, h):
    xz = h @ lp['in_w'] + lp['in_b']
    xm, z = jnp.split(xz, 2, axis=-1)
    n = xm.shape[0]
    xpad = jnp.pad(xm, ((D_CONV - 1, 0), (0, 0)))
    conv = jnp.zeros_like(xm)
    for k in range(D_CONV):
        conv = conv + xpad[k:k + n] * lp['conv_w'][:, k][None, :]
    xc = _silu(conv + lp['conv_b'][None, :])
    xdbl = xc @ lp['xproj_w']
    dt = jax.nn.softplus(xdbl[:, :DT_RANK] @ lp['dt_w'] + lp['dt_b'])
    b = xdbl[:, DT_RANK:DT_RANK + D_STATE]
    c = xdbl[:, DT_RANK + D_STATE:]
    a = -jnp.exp(lp['A_log'])
    return _mamba_scan(xc, dt, b, c, z, a, lp['D'], lp['out_w'], lp['out_b'])


EBLK = 1000


def _edge_body(hs_ref, hd_ref, ea_ref, ws_ref, wd_ref, wr_ref, we_ref,
               b1_ref, w2_ref, b2_ref, out_ref):
    hs = hs_ref[...]
    hd = hd_ref[...]
    dx = hs[:, HID:TBL] - hd[:, HID:TBL]
    radial = jnp.sum(dx * dx, axis=1, keepdims=True)
    z = (jnp.dot(hs[:, :HID], ws_ref[...], preferred_element_type=jnp.float32)
         + jnp.dot(hd[:, :HID], wd_ref[...], preferred_element_type=jnp.float32)
         + jnp.dot(ea_ref[...], we_ref[...], preferred_element_type=jnp.float32)
         + radial * wr_ref[...] + b1_ref[...])
    ef = _silu(z)
    out_ref[...] = _silu(
        jnp.dot(ef, w2_ref[...], preferred_element_type=jnp.float32)
        + b2_ref[...])


def _edge_mlp(hs, hd, ea, lp):
    w1 = lp['edge_w1']
    ws, wd = w1[:HID], w1[HID:2 * HID]
    wr = w1[2 * HID:2 * HID + 1]
    we = w1[2 * HID + 1:]
    b1 = lp['edge_b1'].reshape(1, HID)
    w2 = lp['edge_w2']
    b2 = lp['edge_b2'].reshape(1, HID)
    grid = (E // EBLK,)
    full = lambda i: (0, 0)
    return pl.pallas_call(
        _edge_body,
        grid=grid,
        in_specs=[
            pl.BlockSpec((EBLK, TBL), lambda i: (i, 0)),
            pl.BlockSpec((EBLK, TBL), lambda i: (i, 0)),
            pl.BlockSpec((EBLK, D_EDGE), lambda i: (i, 0)),
            pl.BlockSpec((HID, HID), full),
            pl.BlockSpec((HID, HID), full),
            pl.BlockSpec((1, HID), full),
            pl.BlockSpec((D_EDGE, HID), full),
            pl.BlockSpec((1, HID), full),
            pl.BlockSpec((HID, HID), full),
            pl.BlockSpec((1, HID), full),
        ],
        out_specs=pl.BlockSpec((EBLK, HID), lambda i: (i, 0)),
        out_shape=jax.ShapeDtypeStruct((E, HID), jnp.float32),
    )(hs, hd, ea, ws, wd, wr, we, b1, w2, b2)


def _layer(lp, h, h0, table, idx_all, src_pad, zeros_nh, edge_attr):
    gathered = _sc_gather(table, idx_all)
    hs = gathered[:E]
    hd = gathered[E:2 * E]
    ef = _edge_mlp(hs, hd, edge_attr, lp)
    ef_pad = jnp.concatenate(
        [ef, jnp.zeros((S_PAD - E, HID), jnp.float32)], axis=0)
    parts = _sc_segment_sum(ef_pad, src_pad, zeros_nh)
    mi = parts[0] + parts[1]
    hc = jnp.concatenate([mi, h, h0], axis=1)
    hn = _silu(hc @ lp['node_w1'] + lp['node_b1'])
    hn = hn @ lp['node_w2'] + lp['node_b2']
    mu = hn.mean(axis=-1, keepdims=True)
    var = ((hn - mu) ** 2).mean(axis=-1, keepdims=True)
    hn = (hn - mu) / jnp.sqrt(var + 1e-05) * lp['ln_g'] + lp['ln_b']
    hn = jnp.clip(hn, -10.0, 10.0)
    return _mamba(lp, hn)


def kernel(h0, x, edges, edge_attr, node_mask, edge_mask, n_nodes, params):
    src = edges[0].astype(jnp.int32)
    dst = edges[1].astype(jnp.int32)
    idx_all = jnp.concatenate(
        [src, dst, jnp.zeros((G_PAD - 2 * E,), jnp.int32)], axis=0)
    src_pad = jnp.concatenate(
        [src, jnp.zeros((S_PAD - E,), jnp.int32)], axis=0)
    x_pad = jnp.pad(x, ((0, 0), (0, TBL - HID - x.shape[1])))
    zeros_nh = jnp.zeros((N, HID), jnp.float32)
    h = h0 @ params['emb_w'] + params['emb_b']
    for lp in params['layers']:
        table = jnp.concatenate([h, x_pad], axis=1)
        h = _layer(lp, h, h0, table, idx_all, src_pad, zeros_nh, edge_attr)
    h = _silu(h @ params['dec_w1'] + params['dec_b1'])
    h = h @ params['dec_w2'] + params['dec_b2']
    h = h * node_mask
    h = h.reshape(-1, N, HID).sum(axis=1)
    g = _silu(h @ params['g_w1'] + params['g_b1'])
    pred = g @ params['g_w2'] + params['g_b2']
    return pred[:, 0]
